# Initial kernel scaffold; baseline (speedup 1.0000x reference)
import jax
import jax.numpy as jnp
from jax.experimental import pallas as pl


def kernel(x, index):
    raise NotImplementedError("write your pallas kernel here")


# speedup vs baseline: 3.5032x; 3.5032x over previous
import functools

import jax
import jax.numpy as jnp
from jax import lax
from jax.experimental import pallas as pl
from jax.experimental.pallas import tpu as pltpu
from jax.experimental.pallas import tpu_sc as plsc

N_ROWS = 160000
N_COLS = 256
N_SEG = 10000
S_PAD = 10240
NC = 2
NS = 16
L = 16
DC = N_COLS // NC
RPT = N_ROWS // NS
CH = 80
NCH = RPT // CH
SEG_PT = S_PAD // NS
OB = 64
NOB = SEG_PT // OB
CNTW = 16


def _scatter_mean_body(x_hbm, idx_hbm, out_hbm,
                       acc_sh, cnt_sh, x_buf, idx_buf, ones_buf, obuf, cbuf):
    c = lax.axis_index("c")
    s = lax.axis_index("s")
    col0 = c * DC
    row0 = s * RPT
    seg0 = s * SEG_PT

    zv = jnp.zeros((L,), jnp.float32)
    onev = jnp.ones((L,), jnp.float32)

    def fill_ones(i, carry):
        ones_buf[pl.ds(i * L, L)] = onev
        return carry
    lax.fori_loop(0, CH // L, fill_ones, 0)

    def zero_blk(i, carry):
        for jj in range(DC // L):
            obuf[i, pl.ds(jj * L, L)] = zv
        return carry
    lax.fori_loop(0, OB, zero_blk, 0)

    def zero_cnt(i, carry):
        cbuf[pl.ds(i * L, L)] = zv
        return carry
    lax.fori_loop(0, OB // L, zero_cnt, 0)

    for m in range(NOB):
        pltpu.sync_copy(obuf, acc_sh.at[pl.ds(seg0 + m * OB, OB), :])
        pltpu.sync_copy(cbuf, cnt_sh.at[pl.ds(seg0 + m * OB, OB)])
    plsc.subcore_barrier()

    def chunk(k, carry):
        r0 = pl.multiple_of(row0 + k * CH, 8)
        pltpu.sync_copy(idx_hbm.at[pl.ds(r0, CH)], idx_buf)
        pltpu.sync_copy(x_hbm.at[pl.ds(r0, CH), pl.ds(col0, DC)], x_buf)
        pltpu.sync_copy(x_buf, acc_sh.at[idx_buf], add=True)
        pltpu.sync_copy(ones_buf, cnt_sh.at[idx_buf], add=True)
        return carry
    lax.fori_loop(0, NCH, chunk, 0)
    plsc.subcore_barrier()

    def finalize(m, carry):
        g0 = seg0 + m * OB
        pltpu.sync_copy(acc_sh.at[pl.ds(g0, OB), :], obuf)
        pltpu.sync_copy(cnt_sh.at[pl.ds(g0, OB)], cbuf)

        def rowfix(g, inner):
            cv = jnp.maximum(cbuf[pl.ds(g * L, L)], 1.0)
            rv = jnp.full((L,), 1.0, jnp.float32) / cv
            for j in range(L):
                rvec = jnp.full((L,), rv[j], jnp.float32)
                row = g * L + j
                for jj in range(DC // L):
                    sl = pl.ds(jj * L, L)
                    obuf[row, sl] = obuf[row, sl] * rvec
            return inner
        lax.fori_loop(0, OB // L, rowfix, 0)
        pltpu.sync_copy(obuf, out_hbm.at[pl.ds(g0, OB), pl.ds(col0, DC)])
        return carry
    lax.fori_loop(0, NOB, finalize, 0)


@jax.jit
def kernel(x, index):
    idx32 = index.astype(jnp.int32)
    mesh = plsc.VectorSubcoreMesh(core_axis_name="c", subcore_axis_name="s")
    f = pl.kernel(
        _scatter_mean_body,
        out_type=jax.ShapeDtypeStruct((S_PAD, N_COLS), jnp.float32),
        mesh=mesh,
        scratch_types=[
            pltpu.VMEM_SHARED((S_PAD, DC), jnp.float32),
            pltpu.VMEM_SHARED((S_PAD,), jnp.float32),
            pltpu.VMEM((CH, DC), jnp.float32),
            pltpu.VMEM((CH,), jnp.int32),
            pltpu.VMEM((CH,), jnp.float32),
            pltpu.VMEM((OB, DC), jnp.float32),
            pltpu.VMEM((OB,), jnp.float32),
        ],
    )
    return f(x, idx32)[:N_SEG]


# speedup vs baseline: 5.6968x; 1.6262x over previous
import functools

import jax
import jax.numpy as jnp
from jax import lax
from jax.experimental import pallas as pl
from jax.experimental.pallas import tpu as pltpu
from jax.experimental.pallas import tpu_sc as plsc

N_ROWS = 160000
N_COLS = 256
N_SEG = 10000
S_PAD = 10240
NC = 2
NS = 16
L = 16
DC = N_COLS // NC
RPT = N_ROWS // NS
CH = 80
NCH = RPT // CH
SEG_PT = S_PAD // NS
OB = 64
NOB = SEG_PT // OB
CNTW = 16


def _scatter_mean_body(x_hbm, idx_hbm, out_hbm,
                       acc_sh, cnt_sh, x_a, x_b, idx_a, idx_b, ones_buf,
                       obuf, cbuf,
                       sem_ia, sem_xa, sem_ib, sem_xb,
                       sem_sa, sem_oa, sem_sb, sem_ob):
    c = lax.axis_index("c")
    s = lax.axis_index("s")
    col0 = c * DC
    row0 = s * RPT
    seg0 = s * SEG_PT

    zv = jnp.zeros((L,), jnp.float32)
    onev = jnp.ones((L,), jnp.float32)

    def fill_ones(i, carry):
        ones_buf[pl.ds(i * L, L)] = onev
        return carry
    lax.fori_loop(0, CH // L, fill_ones, 0)

    def zero_blk(i, carry):
        for jj in range(DC // L):
            obuf[i, pl.ds(jj * L, L)] = zv
        return carry
    lax.fori_loop(0, OB, zero_blk, 0)

    def zero_cnt(i, carry):
        cbuf[pl.ds(i * L, L)] = zv
        return carry
    lax.fori_loop(0, OB // L, zero_cnt, 0)

    for m in range(NOB):
        pltpu.sync_copy(obuf, acc_sh.at[pl.ds(seg0 + m * OB, OB), :])
        pltpu.sync_copy(cbuf, cnt_sh.at[pl.ds(seg0 + m * OB, OB)])
    plsc.subcore_barrier()

    def start_load(k, idxb, xb, sem_i, sem_x):
        r0 = pl.multiple_of(row0 + k * CH, 8)
        pltpu.async_copy(idx_hbm.at[pl.ds(r0, CH)], idxb, sem_i)
        pltpu.async_copy(x_hbm.at[pl.ds(r0, CH), pl.ds(col0, DC)], xb, sem_x)

    def wait_load(idxb, xb, sem_i, sem_x):
        pltpu.make_async_copy(idx_hbm.at[pl.ds(0, CH)], idxb, sem_i).wait()
        pltpu.make_async_copy(
            x_hbm.at[pl.ds(0, CH), pl.ds(col0, DC)], xb, sem_x).wait()

    def start_scatter(idxb, xb, sem_s, sem_o):
        pltpu.async_copy(xb, acc_sh.at[idxb], sem_s, add=True)
        pltpu.async_copy(ones_buf, cnt_sh.at[idxb], sem_o, add=True)

    def wait_scatter(idxb, xb, sem_s, sem_o):
        pltpu.make_async_copy(xb, acc_sh.at[idxb], sem_s).wait()
        pltpu.make_async_copy(ones_buf, cnt_sh.at[idxb], sem_o).wait()

    pltpu.sync_copy(idx_hbm.at[pl.ds(pl.multiple_of(row0, 8), CH)], idx_a)
    pltpu.sync_copy(
        x_hbm.at[pl.ds(pl.multiple_of(row0, 8), CH), pl.ds(col0, DC)], x_a)
    pltpu.sync_copy(x_a, acc_sh.at[idx_a], add=True)
    pltpu.sync_copy(ones_buf, cnt_sh.at[idx_a], add=True)
    start_load(1, idx_b, x_b, sem_ib, sem_xb)

    NPAIR = (NCH - 1) // 2

    def pair(i, carry):
        k0 = 2 * i + 1
        k1 = 2 * i + 2
        wait_load(idx_b, x_b, sem_ib, sem_xb)

        @pl.when(i > 0)
        def _():
            wait_scatter(idx_a, x_a, sem_sa, sem_oa)
        start_load(k1, idx_a, x_a, sem_ia, sem_xa)
        start_scatter(idx_b, x_b, sem_sb, sem_ob)
        wait_load(idx_a, x_a, sem_ia, sem_xa)
        start_scatter(idx_a, x_a, sem_sa, sem_oa)

        @pl.when(i < NPAIR - 1)
        def _():
            wait_scatter(idx_b, x_b, sem_sb, sem_ob)
            start_load(k0 + 2, idx_b, x_b, sem_ib, sem_xb)
        return carry
    lax.fori_loop(0, NPAIR, pair, 0)
    wait_scatter(idx_a, x_a, sem_sa, sem_oa)
    wait_scatter(idx_b, x_b, sem_sb, sem_ob)
    plsc.subcore_barrier()

    def finalize(m, carry):
        g0 = seg0 + m * OB
        pltpu.sync_copy(acc_sh.at[pl.ds(g0, OB), :], obuf)
        pltpu.sync_copy(cnt_sh.at[pl.ds(g0, OB)], cbuf)

        def rowfix(g, inner):
            cv = jnp.maximum(cbuf[pl.ds(g * L, L)], 1.0)
            rv = jnp.full((L,), 1.0, jnp.float32) / cv
            for j in range(L):
                rvec = jnp.full((L,), rv[j], jnp.float32)
                row = g * L + j
                for jj in range(DC // L):
                    sl = pl.ds(jj * L, L)
                    obuf[row, sl] = obuf[row, sl] * rvec
            return inner
        lax.fori_loop(0, OB // L, rowfix, 0)
        pltpu.sync_copy(obuf, out_hbm.at[pl.ds(g0, OB), pl.ds(col0, DC)])
        return carry
    lax.fori_loop(0, NOB, finalize, 0)


@jax.jit
def kernel(x, index):
    idx32 = index.astype(jnp.int32)
    mesh = plsc.VectorSubcoreMesh(core_axis_name="c", subcore_axis_name="s")
    f = pl.kernel(
        _scatter_mean_body,
        out_type=jax.ShapeDtypeStruct((S_PAD, N_COLS), jnp.float32),
        mesh=mesh,
        scratch_types=[
            pltpu.VMEM_SHARED((S_PAD, DC), jnp.float32),
            pltpu.VMEM_SHARED((S_PAD,), jnp.float32),
            pltpu.VMEM((CH, DC), jnp.float32),
            pltpu.VMEM((CH, DC), jnp.float32),
            pltpu.VMEM((CH,), jnp.int32),
            pltpu.VMEM((CH,), jnp.int32),
            pltpu.VMEM((CH,), jnp.float32),
            pltpu.VMEM((OB, DC), jnp.float32),
            pltpu.VMEM((OB,), jnp.float32),
        ] + [pltpu.SemaphoreType.DMA] * 8,
    )
    return f(x, idx32)[:N_SEG]


# speedup vs baseline: 5.7371x; 1.0071x over previous
import functools

import jax
import jax.numpy as jnp
from jax import lax
from jax.experimental import pallas as pl
from jax.experimental.pallas import tpu as pltpu
from jax.experimental.pallas import tpu_sc as plsc

N_ROWS = 160000
N_COLS = 256
N_SEG = 10000
S_PAD = 10240
NC = 2
NS = 16
L = 16
DC = N_COLS // NC
RPT = N_ROWS // NS
CH = 80
NCH = RPT // CH
SEG_PT = S_PAD // NS
OB = 64
NOB = SEG_PT // OB
CNTW = 16


def _scatter_mean_body(x_hbm, idx_hbm, out_hbm,
                       acc_sh, cnt_sh, x_a, x_b, idx_a, idx_b, ones_buf,
                       obuf, cbuf,
                       sem_ia, sem_xa, sem_ib, sem_xb, sem_oa, sem_ob):
    c = lax.axis_index("c")
    s = lax.axis_index("s")
    col0 = c * DC
    row0 = s * RPT
    seg0 = s * SEG_PT

    zv = jnp.zeros((L,), jnp.float32)
    onev = jnp.ones((L,), jnp.float32)

    def fill_ones(i, carry):
        ones_buf[pl.ds(i * L, L)] = onev
        return carry
    lax.fori_loop(0, CH // L, fill_ones, 0)

    def zero_blk(i, carry):
        for jj in range(DC // L):
            obuf[i, pl.ds(jj * L, L)] = zv
        return carry
    lax.fori_loop(0, OB, zero_blk, 0)

    def zero_cnt(i, carry):
        cbuf[pl.ds(i * L, L)] = zv
        return carry
    lax.fori_loop(0, OB // L, zero_cnt, 0)

    for m in range(NOB):
        pltpu.sync_copy(obuf, acc_sh.at[pl.ds(seg0 + m * OB, OB), :])
        pltpu.sync_copy(cbuf, cnt_sh.at[pl.ds(seg0 + m * OB, OB)])
    plsc.subcore_barrier()

    def start_load(k, idxb, xb, sem_i, sem_x):
        r0 = pl.multiple_of(row0 + k * CH, 8)
        pltpu.async_copy(idx_hbm.at[pl.ds(r0, CH)], idxb, sem_i)
        pltpu.async_copy(x_hbm.at[pl.ds(r0, CH), pl.ds(col0, DC)], xb, sem_x)

    def wait_load(idxb, xb, sem_i, sem_x):
        pltpu.make_async_copy(idx_hbm.at[pl.ds(0, CH)], idxb, sem_i).wait()
        pltpu.make_async_copy(
            x_hbm.at[pl.ds(0, CH), pl.ds(col0, DC)], xb, sem_x).wait()

    def scatter(idxb, xb, sem_o):
        d = pltpu.async_copy(ones_buf, cnt_sh.at[idxb], sem_o, add=True)
        pltpu.sync_copy(xb, acc_sh.at[idxb], add=True)
        d.wait()

    start_load(0, idx_a, x_a, sem_ia, sem_xa)

    def pair(i, carry):
        k0 = 2 * i
        k1 = 2 * i + 1
        wait_load(idx_a, x_a, sem_ia, sem_xa)
        start_load(k1, idx_b, x_b, sem_ib, sem_xb)
        scatter(idx_a, x_a, sem_oa)
        wait_load(idx_b, x_b, sem_ib, sem_xb)
        start_load(k1 + 1, idx_a, x_a, sem_ia, sem_xa)
        scatter(idx_b, x_b, sem_ob)
        return carry
    lax.fori_loop(0, (NCH - 1) // 2, pair, 0)
    wait_load(idx_a, x_a, sem_ia, sem_xa)
    scatter(idx_a, x_a, sem_oa)
    plsc.subcore_barrier()

    def finalize(m, carry):
        g0 = seg0 + m * OB
        pltpu.sync_copy(acc_sh.at[pl.ds(g0, OB), :], obuf)
        pltpu.sync_copy(cnt_sh.at[pl.ds(g0, OB)], cbuf)

        def rowfix(g, inner):
            cv = jnp.maximum(cbuf[pl.ds(g * L, L)], 1.0)
            rv = jnp.full((L,), 1.0, jnp.float32) / cv
            for j in range(L):
                rvec = jnp.full((L,), rv[j], jnp.float32)
                row = g * L + j
                for jj in range(DC // L):
                    sl = pl.ds(jj * L, L)
                    obuf[row, sl] = obuf[row, sl] * rvec
            return inner
        lax.fori_loop(0, OB // L, rowfix, 0)
        pltpu.sync_copy(obuf, out_hbm.at[pl.ds(g0, OB), pl.ds(col0, DC)])
        return carry
    lax.fori_loop(0, NOB, finalize, 0)


@jax.jit
def kernel(x, index):
    idx32 = index.astype(jnp.int32)
    mesh = plsc.VectorSubcoreMesh(core_axis_name="c", subcore_axis_name="s")
    f = pl.kernel(
        _scatter_mean_body,
        out_type=jax.ShapeDtypeStruct((S_PAD, N_COLS), jnp.float32),
        mesh=mesh,
        scratch_types=[
            pltpu.VMEM_SHARED((S_PAD, DC), jnp.float32),
            pltpu.VMEM_SHARED((S_PAD,), jnp.float32),
            pltpu.VMEM((CH, DC), jnp.float32),
            pltpu.VMEM((CH, DC), jnp.float32),
            pltpu.VMEM((CH,), jnp.int32),
            pltpu.VMEM((CH,), jnp.int32),
            pltpu.VMEM((CH,), jnp.float32),
            pltpu.VMEM((OB, DC), jnp.float32),
            pltpu.VMEM((OB,), jnp.float32),
        ] + [pltpu.SemaphoreType.DMA] * 6,
    )
    return f(x, idx32)[:N_SEG]


# speedup vs baseline: 7.3595x; 1.2828x over previous
import jax
import jax.numpy as jnp
from jax import lax
from jax.experimental import pallas as pl
from jax.experimental.pallas import tpu as pltpu
from jax.experimental.pallas import tpu_sc as plsc

N_ROWS = 160000
N_COLS = 256
N_SEG = 10000
S_PAD = 10240
NC = 2
NS = 16
L = 16
DC = N_COLS // NC
RPT = N_ROWS // NS
CH = 80
NCH = RPT // CH
ICH = 128
NBUF = 3
SEG_PT = S_PAD // NS
OB = 80
NOB = SEG_PT // OB


def _scatter_mean_body(x_hbm, idx_hbm, out_hbm,
                       acc_sh, cnt_sh, idx_all, ones_buf, cbuf,
                       x_bufs, x_sems):
    c = lax.axis_index("c")
    s = lax.axis_index("s")
    col0 = c * DC
    row0 = s * RPT
    seg0 = s * SEG_PT

    zv = jnp.zeros((L,), jnp.float32)
    onev = jnp.ones((L,), jnp.float32)

    def fill_ones(i, carry):
        ones_buf[pl.ds(i * L, L)] = onev
        return carry
    lax.fori_loop(0, CH // L, fill_ones, 0)

    obuf = x_bufs[0]

    def zero_blk(i, carry):
        for jj in range(DC // L):
            obuf[i, pl.ds(jj * L, L)] = zv
        return carry
    lax.fori_loop(0, OB, zero_blk, 0)

    def zero_cnt(i, carry):
        cbuf[pl.ds(i * L, L)] = zv
        return carry
    lax.fori_loop(0, OB // L, zero_cnt, 0)

    for m in range(NOB):
        pltpu.sync_copy(obuf, acc_sh.at[pl.ds(seg0 + m * OB, OB), :])
        pltpu.sync_copy(cbuf, cnt_sh.at[pl.ds(seg0 + m * OB, OB)])

    pltpu.sync_copy(idx_hbm.at[pl.ds(pl.multiple_of(s * ICH, 8), ICH), :],
                    idx_all)
    plsc.subcore_barrier()

    def start_load(k, b):
        r0 = pl.multiple_of(row0 + k * CH, 8)
        pltpu.async_copy(x_hbm.at[pl.ds(r0, CH), pl.ds(col0, DC)],
                         x_bufs[b], x_sems[b])

    def wait_load(b):
        pltpu.make_async_copy(
            x_hbm.at[pl.ds(0, CH), pl.ds(col0, DC)],
            x_bufs[b], x_sems[b]).wait()

    def scatter(k, b):
        d = pltpu.async_copy(ones_buf, cnt_sh.at[idx_all.at[k]],
                             x_sems[NBUF], add=True)
        pltpu.sync_copy(x_bufs[b], acc_sh.at[idx_all.at[k]], add=True)
        d.wait()

    for b in range(NBUF):
        start_load(b, b)

    NTRI = (NCH - 2) // NBUF

    def tri(i, carry):
        for j in range(NBUF):
            k = NBUF * i + j
            wait_load(j)
            scatter(k, j)

            @pl.when(k + NBUF < NCH)
            def _():
                start_load(k + NBUF, j)
        return carry
    lax.fori_loop(0, NTRI, tri, 0)
    for j in range(NTRI * NBUF, NCH):
        b = j % NBUF
        wait_load(b)
        scatter(j, b)
    plsc.subcore_barrier()

    def finalize(m, carry):
        g0 = seg0 + m * OB
        pltpu.sync_copy(acc_sh.at[pl.ds(g0, OB), :], obuf)
        pltpu.sync_copy(cnt_sh.at[pl.ds(g0, OB)], cbuf)

        def rowfix(g, inner):
            cv = jnp.maximum(cbuf[pl.ds(g * L, L)], 1.0)
            rv = jnp.full((L,), 1.0, jnp.float32) / cv
            for j in range(L):
                rvec = jnp.full((L,), rv[j], jnp.float32)
                row = g * L + j
                for jj in range(DC // L):
                    sl = pl.ds(jj * L, L)
                    obuf[row, sl] = obuf[row, sl] * rvec
            return inner
        lax.fori_loop(0, OB // L, rowfix, 0)
        pltpu.sync_copy(obuf, out_hbm.at[pl.ds(g0, OB), pl.ds(col0, DC)])
        return carry
    lax.fori_loop(0, NOB, finalize, 0)


@jax.jit
def kernel(x, index):
    idx32 = index.astype(jnp.int32)
    idx2d = jnp.pad(idx32.reshape(NS, NCH, CH),
                    ((0, 0), (0, ICH - NCH), (0, 0)))
    idx2d = idx2d.reshape(NS * ICH, CH)
    mesh = plsc.VectorSubcoreMesh(core_axis_name="c", subcore_axis_name="s")
    f = pl.kernel(
        _scatter_mean_body,
        out_type=jax.ShapeDtypeStruct((S_PAD, N_COLS), jnp.float32),
        mesh=mesh,
        scratch_types=[
            pltpu.VMEM_SHARED((S_PAD, DC), jnp.float32),
            pltpu.VMEM_SHARED((S_PAD,), jnp.float32),
            pltpu.VMEM((ICH, CH), jnp.int32),
            pltpu.VMEM((CH,), jnp.float32),
            pltpu.VMEM((OB,), jnp.float32),
            [pltpu.VMEM((CH, DC), jnp.float32)] * NBUF,
            [pltpu.SemaphoreType.DMA] * (NBUF + 1),
        ],
    )
    return f(x, idx2d)[:N_SEG]


# speedup vs baseline: 7.4298x; 1.0096x over previous
import jax
import jax.numpy as jnp
from jax import lax
from jax.experimental import pallas as pl
from jax.experimental.pallas import tpu as pltpu
from jax.experimental.pallas import tpu_sc as plsc

N_ROWS = 160000
N_COLS = 256
N_SEG = 10000
S_PAD = 10240
NC = 2
NS = 16
L = 16
DC = N_COLS // NC
RPT = N_ROWS // NS
CH = 80
NCH = RPT // CH
NBUF = 4
SEG_PT = S_PAD // NS
OB = 80
NOB = SEG_PT // OB


def _scatter_mean_body(x_hbm, idx_hbm, out_hbm,
                       acc_sh, cnt_sh, ones_buf, cbuf,
                       x_bufs, idx_bufs, x_sems, i_sems):
    c = lax.axis_index("c")
    s = lax.axis_index("s")
    col0 = c * DC
    row0 = s * RPT
    seg0 = s * SEG_PT

    zv = jnp.zeros((L,), jnp.float32)
    onev = jnp.ones((L,), jnp.float32)

    def fill_ones(i, carry):
        ones_buf[pl.ds(i * L, L)] = onev
        return carry
    lax.fori_loop(0, CH // L, fill_ones, 0)

    obuf = x_bufs[0]

    def zero_blk(i, carry):
        for jj in range(DC // L):
            obuf[i, pl.ds(jj * L, L)] = zv
        return carry
    lax.fori_loop(0, OB, zero_blk, 0)

    def zero_cnt(i, carry):
        cbuf[pl.ds(i * L, L)] = zv
        return carry
    lax.fori_loop(0, OB // L, zero_cnt, 0)

    for m in range(NOB):
        pltpu.sync_copy(obuf, acc_sh.at[pl.ds(seg0 + m * OB, OB), :])
        pltpu.sync_copy(cbuf, cnt_sh.at[pl.ds(seg0 + m * OB, OB)])

    plsc.subcore_barrier()

    def start_load(k, b):
        r0 = pl.multiple_of(row0 + k * CH, 8)
        pltpu.async_copy(idx_hbm.at[pl.ds(r0, CH)], idx_bufs[b], i_sems[b])
        pltpu.async_copy(x_hbm.at[pl.ds(r0, CH), pl.ds(col0, DC)],
                         x_bufs[b], x_sems[b])

    def wait_load(b):
        pltpu.make_async_copy(idx_hbm.at[pl.ds(0, CH)],
                              idx_bufs[b], i_sems[b]).wait()
        pltpu.make_async_copy(
            x_hbm.at[pl.ds(0, CH), pl.ds(col0, DC)],
            x_bufs[b], x_sems[b]).wait()

    def scatter(k, b):
        d = pltpu.async_copy(ones_buf, cnt_sh.at[idx_bufs[b]],
                             x_sems[NBUF], add=True)
        pltpu.sync_copy(x_bufs[b], acc_sh.at[idx_bufs[b]], add=True)
        d.wait()

    for b in range(NBUF):
        start_load(b, b)

    NTRI = NCH // NBUF

    def tri(i, carry):
        for j in range(NBUF):
            k = NBUF * i + j
            wait_load(j)
            scatter(k, j)

            @pl.when(k + NBUF < NCH)
            def _():
                start_load(k + NBUF, j)
        return carry
    lax.fori_loop(0, NTRI, tri, 0)
    for j in range(NTRI * NBUF, NCH):
        b = j % NBUF
        wait_load(b)
        scatter(j, b)
    plsc.subcore_barrier()

    def finalize(m, carry):
        g0 = seg0 + m * OB
        pltpu.sync_copy(acc_sh.at[pl.ds(g0, OB), :], obuf)
        pltpu.sync_copy(cnt_sh.at[pl.ds(g0, OB)], cbuf)

        def rowfix(g, inner):
            cv = jnp.maximum(cbuf[pl.ds(g * L, L)], 1.0)
            rv = jnp.full((L,), 1.0, jnp.float32) / cv
            for j in range(L):
                rvec = jnp.full((L,), rv[j], jnp.float32)
                row = g * L + j
                for jj in range(DC // L):
                    sl = pl.ds(jj * L, L)
                    obuf[row, sl] = obuf[row, sl] * rvec
            return inner
        lax.fori_loop(0, OB // L, rowfix, 0)
        pltpu.sync_copy(obuf, out_hbm.at[pl.ds(g0, OB), pl.ds(col0, DC)])
        return carry
    lax.fori_loop(0, NOB, finalize, 0)


@jax.jit
def kernel(x, index):
    idx32 = index.astype(jnp.int32)
    mesh = plsc.VectorSubcoreMesh(core_axis_name="c", subcore_axis_name="s")
    f = pl.kernel(
        _scatter_mean_body,
        out_type=jax.ShapeDtypeStruct((S_PAD, N_COLS), jnp.float32),
        mesh=mesh,
        scratch_types=[
            pltpu.VMEM_SHARED((S_PAD, DC), jnp.float32),
            pltpu.VMEM_SHARED((S_PAD,), jnp.float32),
            pltpu.VMEM((CH,), jnp.float32),
            pltpu.VMEM((OB,), jnp.float32),
            [pltpu.VMEM((CH, DC), jnp.float32)] * NBUF,
            [pltpu.VMEM((CH,), jnp.int32)] * NBUF,
            [pltpu.SemaphoreType.DMA] * (NBUF + 1),
            [pltpu.SemaphoreType.DMA] * NBUF,
        ],
    )
    return f(x, idx32)[:N_SEG]
